# Initial kernel scaffold; baseline (speedup 1.0000x reference)
import jax
import jax.numpy as jnp
from jax.experimental import pallas as pl


def kernel(task_x, proc_x, edge_attr, params, edge_index, task_batch, proc_batch):
    raise NotImplementedError("write your pallas kernel here")


# speedup vs baseline: 27.8509x; 27.8509x over previous
import functools

import jax
import jax.numpy as jnp
from jax import lax
from jax.experimental import pallas as pl
from jax.experimental.pallas import tpu as pltpu
from jax.experimental.pallas import tpu_sc as plsc

H = 2
C = 128
N = 10000
E = 160000
ED = 16
NSUB = 16
RW = 144
EPT = E // NSUB
CH = 80
NCHUNK = EPT // CH
BLK = 2000
NP = 10240

f32 = jnp.float32
i32 = jnp.int32


def _mm_bias_body(x_ref, w_ref, b_ref, o_ref):
    o_ref[...] = (
        jnp.dot(x_ref[...], w_ref[...], preferred_element_type=f32) + b_ref[...]
    )


def _matmul_bias(x, w, b, block=BLK):
    m, k = x.shape
    n = w.shape[1]
    return pl.pallas_call(
        _mm_bias_body,
        grid=(m // block,),
        in_specs=[
            pl.BlockSpec((block, k), lambda i: (i, 0)),
            pl.BlockSpec((k, n), lambda i: (0, 0)),
            pl.BlockSpec((1, n), lambda i: (0, 0)),
        ],
        out_specs=pl.BlockSpec((block, n), lambda i: (i, 0)),
        out_shape=jax.ShapeDtypeStruct((m, n), f32),
    )(x, w, b.reshape(1, n))


def _xs_body(x_ref, w_ref, o_ref):
    o_ref[...] = jnp.dot(x_ref[...], w_ref[0], preferred_element_type=f32)


def _xs_stacked(x, w_lin):
    nb = N // BLK
    w3 = w_lin.reshape(C, H, C).transpose(1, 0, 2)
    return pl.pallas_call(
        _xs_body,
        grid=(H, nb),
        in_specs=[
            pl.BlockSpec((BLK, C), lambda h, i: (i, 0)),
            pl.BlockSpec((1, C, C), lambda h, i: (h, 0, 0)),
        ],
        out_specs=pl.BlockSpec((BLK, C), lambda h, i: (h * nb + i, 0)),
        out_shape=jax.ShapeDtypeStruct((H * N, C), f32),
    )(x, w3)


def _post_body(x_ref, acc_ref, xs_ref, s_ref, wp_ref, b2_ref, g_ref, be_ref, o_ref):
    svals = s_ref[...]
    proj = b2_ref[...]
    for h in range(H):
        inv = svals[:, h:h + 1]
        slw = svals[:, H + h:H + h + 1]
        out_h = acc_ref[h] * inv + xs_ref[h] * slw
        proj = proj + jnp.dot(out_h, wp_ref[h], preferred_element_type=f32)
    hh = x_ref[...] + jnp.where(proj > 0, proj, jnp.exp(jnp.minimum(proj, 0.0)) - 1.0)
    mu = jnp.mean(hh, axis=-1, keepdims=True)
    xc = hh - mu
    var = jnp.mean(xc * xc, axis=-1, keepdims=True)
    o_ref[...] = xc * lax.rsqrt(var + 1e-5) * g_ref[...] + be_ref[...]


def _layer_post(x, accm, xs3, s_pad, wp2, b2, gamma, beta):
    return pl.pallas_call(
        _post_body,
        grid=(N // BLK,),
        in_specs=[
            pl.BlockSpec((BLK, C), lambda i: (i, 0)),
            pl.BlockSpec((H, BLK, C), lambda i: (0, i, 0)),
            pl.BlockSpec((H, BLK, C), lambda i: (0, i, 0)),
            pl.BlockSpec((BLK, C), lambda i: (i, 0)),
            pl.BlockSpec((H, C, C), lambda i: (0, 0, 0)),
            pl.BlockSpec((1, C), lambda i: (0, 0)),
            pl.BlockSpec((1, C), lambda i: (0, 0)),
            pl.BlockSpec((1, C), lambda i: (0, 0)),
        ],
        out_specs=pl.BlockSpec((BLK, C), lambda i: (i, 0)),
        out_shape=jax.ShapeDtypeStruct((N, C), f32),
    )(x, accm, xs3, s_pad, wp2, b2.reshape(1, C), gamma.reshape(1, C),
      beta.reshape(1, C))


def _final_body(t_ref, w1_ref, b1_ref, w2_ref, lo_ref, sum_ref):
    i = pl.program_id(0)
    h1 = jnp.dot(t_ref[...], w1_ref[...], preferred_element_type=f32) + b1_ref[...]
    h1 = jnp.maximum(h1, 0.0)
    lo_ref[...] = jnp.sum(h1 * w2_ref[...], axis=1, keepdims=True).reshape(1, 1, -1)
    part = jnp.sum(t_ref[...], axis=0, keepdims=True)

    @pl.when(i == 0)
    def _():
        sum_ref[...] = part

    @pl.when(i > 0)
    def _():
        sum_ref[...] += part


def _final_head(t, pt1, pt1b, pt2):
    k = pt1.shape[1]
    return pl.pallas_call(
        _final_body,
        grid=(N // BLK,),
        in_specs=[
            pl.BlockSpec((BLK, C), lambda i: (i, 0)),
            pl.BlockSpec((C, k), lambda i: (0, 0)),
            pl.BlockSpec((1, k), lambda i: (0, 0)),
            pl.BlockSpec((1, k), lambda i: (0, 0)),
        ],
        out_specs=[
            pl.BlockSpec((1, 1, BLK), lambda i: (i, 0, 0)),
            pl.BlockSpec((1, C), lambda i: (0, 0)),
        ],
        out_shape=[
            jax.ShapeDtypeStruct((N // BLK, 1, BLK), f32),
            jax.ShapeDtypeStruct((1, C), f32),
        ],
    )(t, pt1, pt1b.reshape(1, k), pt2.reshape(1, k))


def _sc_edge_body(src_h, dst_h, ae_h, asrc_h, adst_h, smax_h, xs_h, acc_o, den_o,
                  acc_sp, den_sp, asrc_v, adst_v, smax_v, sidx_v, didx_v, dhi_v,
                  ae_v, rows_v, oh_v, sem):
    c = lax.axis_index("c")
    s = lax.axis_index("s")
    zero16 = jnp.zeros((16,), f32)

    def zr(i, _):
        for j in range(C // 16):
            rows_v[i, pl.ds(16 * j, 16)] = zero16
            oh_v[i, pl.ds(16 * j, 16)] = zero16
        return 0

    lax.fori_loop(0, CH, zr, 0)

    for b in range(8):
        pltpu.sync_copy(rows_v, acc_sp.at[pl.ds(s * 640 + b * CH, CH)])

    @pl.when(s < 10)
    def _():
        pltpu.sync_copy(rows_v.at[pl.ds(0, 8)], den_sp.at[pl.ds(s * 8, 8)])

    pltpu.sync_copy(asrc_h.at[pl.ds(c * N, N)], asrc_v)
    pltpu.sync_copy(adst_h.at[pl.ds(c * N, N)], adst_v)
    pltpu.sync_copy(smax_h.at[pl.ds(c * 16, 16)], smax_v)
    plsc.subcore_barrier()

    coff = c * N
    base = s * EPT
    iot = lax.iota(i32, 16)
    sv = smax_v[...]
    zc0 = jnp.zeros((16,), i32)

    def chunk(it, carry):
        off = base + it * CH
        pltpu.sync_copy(src_h.at[pl.ds(off, CH)], sidx_v)
        pltpu.sync_copy(dst_h.at[pl.ds(off, CH)], didx_v)
        pltpu.sync_copy(ae_h.at[pl.ds(c * E + off, CH)], ae_v)
        for g in range(CH // 16):
            sl = pl.ds(16 * g, 16)
            sidx_v[sl] = sidx_v[sl] + coff
        cp = pltpu.async_copy(xs_h.at[sidx_v], rows_v, sem)
        exs = []
        cols = []
        for g in range(CH // 16):
            sl = pl.ds(16 * g, 16)
            s16 = sidx_v[sl] - coff
            d16 = didx_v[sl]
            asv = plsc.load_gather(asrc_v, [s16])
            adv = plsc.load_gather(adst_v, [d16])
            mb = adv + sv
            mv = jnp.where(mb >= 0, mb, 0.2 * mb)
            al = asv + adv + ae_v[sl]
            al = jnp.where(al >= 0, al, 0.2 * al)
            ex = jnp.exp(al - mv)
            exs.append(ex)
            cols.append(d16 & 127)
            dhi_v[sl] = d16 >> 7
        for g in range(CH // 16):
            plsc.store_scatter(oh_v, [iot + 16 * g, carry[g]], zero16)
        for g in range(CH // 16):
            plsc.store_scatter(oh_v, [iot + 16 * g, cols[g]], exs[g])
        cp.wait()
        for g in range(CH // 16):
            for l in range(16):
                r = 16 * g + l
                w = exs[g][l]
                for j in range(C // 16):
                    sl = pl.ds(16 * j, 16)
                    rows_v[r, sl] = rows_v[r, sl] * w
        pltpu.sync_copy(rows_v, acc_sp.at[didx_v], add=True)
        pltpu.sync_copy(oh_v, den_sp.at[dhi_v], add=True)
        return tuple(cols)

    lax.fori_loop(0, NCHUNK, chunk, (zc0,) * (CH // 16))
    plsc.subcore_barrier()
    pltpu.sync_copy(acc_sp.at[pl.ds(s * 640, 640)], acc_o.at[c, pl.ds(s * 640, 640)])

    @pl.when(s < 10)
    def _():
        pltpu.sync_copy(den_sp.at[pl.ds(s * 8, 8)], den_o.at[c, pl.ds(s * 8, 8)])


def _sc_edge(src, dst, ae_t, asrc_t, adst_t, smax16, xs_st):
    mesh = plsc.VectorSubcoreMesh(core_axis_name="c", subcore_axis_name="s")
    k = pl.kernel(
        _sc_edge_body,
        out_type=(jax.ShapeDtypeStruct((H, NP, C), f32),
                  jax.ShapeDtypeStruct((H, NP // 128, 128), f32)),
        mesh=mesh,
        compiler_params=pltpu.CompilerParams(needs_layout_passes=False),
        scratch_types=[
            pltpu.VMEM_SHARED((NP, C), f32),
            pltpu.VMEM_SHARED((NP // 128, 128), f32),
            pltpu.VMEM((N,), f32),
            pltpu.VMEM((N,), f32),
            pltpu.VMEM((16,), f32),
            pltpu.VMEM((CH,), i32),
            pltpu.VMEM((CH,), i32),
            pltpu.VMEM((CH,), i32),
            pltpu.VMEM((CH,), f32),
            pltpu.VMEM((CH, C), f32),
            pltpu.VMEM((CH, 128), f32),
            pltpu.SemaphoreType.DMA,
        ],
    )
    return k(src, dst, ae_t, asrc_t, adst_t, smax16, xs_st)


def _lrelu(v):
    return jnp.where(v >= 0, v, 0.2 * v)


def kernel(task_x, proc_x, edge_attr, params, edge_index, task_batch, proc_batch):
    src = edge_index[0]
    dst = edge_index[1]
    ea_mean = jnp.mean(edge_attr, axis=0)

    t = _matmul_bias(task_x, params['Wt'], params['bt'])
    pr = proc_x @ params['Wp'] + params['bp']

    for p in params['layers']:
        x = t
        w3 = p['W_lin'].reshape(C, H, C)
        Wsrc = jnp.einsum('chj,hj->ch', w3, p['att_src'][0])
        Wdst = jnp.einsum('chj,hj->ch', w3, p['att_dst'][0])
        We = jnp.einsum('dhj,hj->dh', p['W_edge'].reshape(ED, H, C), p['att_edge'][0])

        xs_st = _xs_stacked(x, p['W_lin'])
        a_src = x @ Wsrc
        a_dst = x @ Wdst
        ae = edge_attr @ We
        ae_mean = ea_mean @ We
        Smax = jnp.max(a_src, axis=0) + jnp.maximum(jnp.max(ae, axis=0), ae_mean)
        M = _lrelu(a_dst + Smax[None, :])
        exloop = jnp.exp(_lrelu(a_src + a_dst + ae_mean[None, :]) - M)

        smax16 = jnp.repeat(Smax[:, None], 16, axis=1).reshape(-1)
        acc, den2 = _sc_edge(src, dst, ae.T.reshape(-1), a_src.T.reshape(-1),
                             a_dst.T.reshape(-1), smax16, xs_st)
        den = den2.reshape(H, NP)[:, :N].T + exloop
        inv_den = 1.0 / den
        s_pad = jnp.concatenate(
            [inv_den, exloop * inv_den, jnp.zeros((N, C - 2 * H), f32)], axis=1)
        accm = acc[:, :N, :]
        xs3 = xs_st.reshape(H, N, C)
        b2 = p['bias'] @ p['W_proj'] + p['b_proj']
        wp2 = p['W_proj'].reshape(H, C, C)
        t = _layer_post(x, accm, xs3, s_pad, wp2, b2, p['gamma'], p['beta'])

    lo2, tsum = _final_head(t, params['pt1'], params['pt1b'], params['pt2'])
    task_logits = lo2.reshape(N) + params['pt2b'][0]
    task_pool = tsum / N
    proc_pool = jnp.mean(pr, axis=0, keepdims=True)
    g = jnp.concatenate([task_pool, proc_pool], axis=1)
    value = jax.nn.relu(g @ params['v1'] + params['v1b']) @ params['v2'] + params['v2b']
    return (task_logits, value, t, pr)
